# Initial kernel scaffold; baseline (speedup 1.0000x reference)
import jax
import jax.numpy as jnp
from jax.experimental import pallas as pl


def kernel(pred_logits, pred_obj, pred_boxes, target_sizes):
    raise NotImplementedError("write your pallas kernel here")


# speedup vs baseline: 4.1246x; 4.1246x over previous
import functools

import jax
import jax.numpy as jnp
from jax.experimental import pallas as pl
from jax.experimental.pallas import tpu as pltpu

NUM_VALID = 81
K = 100


def _rowmax_body(lg_ref, ob_ref, s_ref):
    x = lg_ref[0]
    rb, c = x.shape
    cls = jax.lax.broadcasted_iota(jnp.int32, (rb, c), 1)
    m = jnp.max(jnp.where(cls < NUM_VALID, x, -jnp.inf), axis=1)
    s_ref[0, 0] = jnp.exp(-ob_ref[0, 0]) * jax.nn.sigmoid(m)


def _row_scores(pred_logits, pred_obj):
    B, N, C = pred_logits.shape
    RB = 1000
    G = (B * N) // RB
    lg = pred_logits.reshape(G, RB, C)
    ob = pred_obj.reshape(G, 1, RB)
    s = pl.pallas_call(
        _rowmax_body,
        grid=(G,),
        in_specs=[
            pl.BlockSpec((1, RB, C), lambda i: (i, 0, 0)),
            pl.BlockSpec((1, 1, RB), lambda i: (i, 0, 0)),
        ],
        out_specs=pl.BlockSpec((1, 1, RB), lambda i: (i, 0, 0)),
        out_shape=jax.ShapeDtypeStruct((G, 1, RB), jnp.float32),
    )(lg, ob)
    return s.reshape(B, N)


def kernel(pred_logits, pred_obj, pred_boxes, target_sizes):
    B, N, C = pred_logits.shape
    s = _row_scores(pred_logits, pred_obj)

    _, rowidx = jax.lax.top_k(s, K)

    glg = jnp.take_along_axis(pred_logits, rowidx[:, :, None], axis=1)
    gob = jnp.take_along_axis(pred_obj, rowidx, axis=1)
    cls = jnp.arange(C)
    probs = jnp.exp(-gob)[:, :, None] * jax.nn.sigmoid(
        jnp.where(cls < NUM_VALID, glg, -jnp.inf)
    )
    scores, pos = jax.lax.top_k(probs.reshape(B, K * C), K)
    labels = pos % C
    boxrow = jnp.take_along_axis(rowidx, pos // C, axis=1)

    gb = jnp.take_along_axis(pred_boxes, boxrow[:, :, None], axis=1)
    cx, cy, w, h = gb[..., 0], gb[..., 1], gb[..., 2], gb[..., 3]
    boxes = jnp.stack(
        [cx - 0.5 * w, cy - 0.5 * h, cx + 0.5 * w, cy + 0.5 * h], axis=-1
    )
    img_h = target_sizes[:, 0]
    img_w = target_sizes[:, 1]
    scale = jnp.stack([img_w, img_h, img_w, img_h], axis=1)
    boxes = boxes * scale[:, None, :]
    return scores, labels, boxes


# speedup vs baseline: 4.1471x; 1.0055x over previous
import functools

import jax
import jax.numpy as jnp
from jax.experimental import pallas as pl
from jax.experimental.pallas import tpu as pltpu

NUM_VALID = 81
K = 100
PACK = 8


def _rowmax_body(lg_ref, ob_ref, s_ref, eo_ref):
    x = lg_ref[...]
    segs = []
    for seg in range(PACK):
        segs.append(jnp.max(x[:, 91 * seg : 91 * seg + NUM_VALID], axis=1))
    m = jnp.stack(segs, axis=1)
    eo = jnp.exp(-ob_ref[...])
    s_ref[...] = eo * jax.nn.sigmoid(m)
    eo_ref[...] = eo


def _row_scores(pred_logits, pred_obj):
    B, N, C = pred_logits.shape
    R = B * N
    RB = 1000
    G = R // (PACK * RB)
    lg = pred_logits.reshape(R // PACK, PACK * C)
    ob = pred_obj.reshape(R // PACK, PACK)
    s, eo = pl.pallas_call(
        _rowmax_body,
        grid=(G,),
        in_specs=[
            pl.BlockSpec((RB, PACK * C), lambda i: (i, 0)),
            pl.BlockSpec((RB, PACK), lambda i: (i, 0)),
        ],
        out_specs=[
            pl.BlockSpec((RB, PACK), lambda i: (i, 0)),
            pl.BlockSpec((RB, PACK), lambda i: (i, 0)),
        ],
        out_shape=[
            jax.ShapeDtypeStruct((R // PACK, PACK), jnp.float32),
            jax.ShapeDtypeStruct((R // PACK, PACK), jnp.float32),
        ],
    )(lg, ob)
    return s.reshape(B, N), eo.reshape(B, N)


def kernel(pred_logits, pred_obj, pred_boxes, target_sizes):
    B, N, C = pred_logits.shape
    s, _ = _row_scores(pred_logits, pred_obj)

    _, rowidx = jax.lax.top_k(s, K)

    glg = jnp.take_along_axis(pred_logits, rowidx[:, :, None], axis=1)
    gob = jnp.take_along_axis(pred_obj, rowidx, axis=1)
    cls = jnp.arange(C)
    probs = jnp.exp(-gob)[:, :, None] * jax.nn.sigmoid(
        jnp.where(cls < NUM_VALID, glg, -jnp.inf)
    )
    scores, pos = jax.lax.top_k(probs.reshape(B, K * C), K)
    labels = pos % C
    boxrow = jnp.take_along_axis(rowidx, pos // C, axis=1)

    gb = jnp.take_along_axis(pred_boxes, boxrow[:, :, None], axis=1)
    cx, cy, w, h = gb[..., 0], gb[..., 1], gb[..., 2], gb[..., 3]
    boxes = jnp.stack(
        [cx - 0.5 * w, cy - 0.5 * h, cx + 0.5 * w, cy + 0.5 * h], axis=-1
    )
    img_h = target_sizes[:, 0]
    img_w = target_sizes[:, 1]
    scale = jnp.stack([img_w, img_h, img_w, img_h], axis=1)
    boxes = boxes * scale[:, None, :]
    return scores, labels, boxes


# speedup vs baseline: 6.0177x; 1.4511x over previous
import functools

import jax
import jax.numpy as jnp
from jax import lax
from jax.experimental import pallas as pl
from jax.experimental.pallas import tpu as pltpu
from jax.experimental.pallas import tpu_sc as plsc

B = 8
N = 5000
C = 91
NUM_VALID = 81
K = 100
RB = 1000
NP = 5120
NVR = NP // 16
NGRP = 16
GV = NVR // NGRP
KP = 112
BIG = 1 << 20


def _rowmax_body(lg_ref, m_ref, lgp_ref):
    x = lg_ref[0]
    cls = jax.lax.broadcasted_iota(jnp.int32, (RB, C), 1)
    m_ref[0, 0] = jnp.max(jnp.where(cls < NUM_VALID, x, -jnp.inf), axis=1)
    lgp_ref[0, :, :C] = x


def _rowmax(pred_logits):
    return pl.pallas_call(
        _rowmax_body,
        grid=(B, N // RB),
        in_specs=[pl.BlockSpec((1, RB, C), lambda b, i: (b, i, 0))],
        out_specs=[
            pl.BlockSpec((1, 1, RB), lambda b, i: (b * (N // RB) + i, 0, 0)),
            pl.BlockSpec((1, RB, 128), lambda b, i: (b * (N // RB) + i, 0, 0)),
        ],
        out_shape=[
            jax.ShapeDtypeStruct((B * N // RB, 1, RB), jnp.float32),
            jax.ShapeDtypeStruct((B * N // RB, RB, 128), jnp.float32),
        ],
    )(pred_logits)


def _postprocess_sc(m_hbm, ob_hbm, lgt_hbm, bx_hbm, scl_hbm,
                    outs, outl, outb,
                    svb, ovb, gmb, rkb, rvb, gib, lrows, pmat, rloc,
                    scob, labb, bxob, bxlb, sclvb, sem):
    wid = lax.axis_index("s") * 2 + lax.axis_index("c")

    @pl.when(wid < B)
    def _():
        b = wid
        IOTA = lax.iota(jnp.int32, 16)

        def splat_f(x):
            return jnp.full((16,), x, jnp.float32)

        def splat_i(x):
            return jnp.full((16,), x, jnp.int32)

        lane0 = IOTA == 0

        pltpu.sync_copy(m_hbm.at[pl.ds(b * NP, NP)], svb)
        pltpu.sync_copy(ob_hbm.at[pl.ds(b * NP, NP)], ovb)
        pltpu.sync_copy(bx_hbm.at[pl.ds(b * (4 * N), 4 * N)], bxlb)
        pltpu.sync_copy(scl_hbm.at[pl.ds(b * 16, 16)], sclvb)

        def init_body(i, _):
            idx = i * 16 + IOTA
            mv = plsc.load_gather(svb, [idx])
            ov = plsc.load_gather(ovb, [idx])
            val = jnp.exp(-ov) * (1.0 / (1.0 + jnp.exp(-mv)))
            val = jnp.where(idx < N, val, -1.0)
            plsc.store_scatter(svb, [idx], val)
            return 0

        lax.fori_loop(0, NVR, init_body, 0)

        def gm_body(g, _):
            def inner(j, acc):
                return jnp.maximum(
                    acc, plsc.load_gather(svb, [(g * GV + j) * 16 + IOTA])
                )

            acc = lax.fori_loop(0, GV, inner, splat_f(-2.0))
            plsc.store_scatter(gmb, [g * 16 + IOTA], acc)
            return 0

        lax.fori_loop(0, NGRP, gm_body, 0)

        def cm_body(g, acc):
            return jnp.maximum(acc, plsc.load_gather(gmb, [g * 16 + IOTA]))

        cm = lax.fori_loop(0, NGRP, cm_body, splat_f(-2.0))

        plsc.store_scatter(rvb, [96 + IOTA], splat_f(-1.0))
        plsc.store_scatter(rkb, [96 + IOTA], splat_i(0))
        plsc.store_scatter(gib, [96 + IOTA], splat_i(b * N))

        def extb_body(k, cm):
            mval = jnp.max(cm)
            L = jnp.min(jnp.where(cm == mval, IOTA, BIG))
            colg = plsc.load_gather(gmb, [IOTA * 16 + L])
            g = jnp.min(jnp.where(colg == mval, IOTA, BIG))
            base = g * GV * 16 + L
            idx1 = base + IOTA * 16
            valid2 = IOTA < (GV - 16)
            idx2 = base + (16 + IOTA) * 16
            c1 = plsc.load_gather(svb, [idx1])
            c2 = plsc.load_gather(svb, [jnp.where(valid2, idx2, 0)])
            c2 = jnp.where(valid2, c2, -2.0)
            j1 = jnp.min(jnp.where(c1 == mval, IOTA, BIG))
            j2 = jnp.min(jnp.where(c2 == mval, 16 + IOTA, BIG))
            j = jnp.minimum(j1, j2)
            r = (g * GV + j) * 16 + L
            plsc.store_scatter(rkb, [splat_i(k)], splat_i(r), mask=lane0)
            plsc.store_scatter(rvb, [splat_i(k)], splat_f(mval), mask=lane0)
            plsc.store_scatter(gib, [splat_i(k)], splat_i(b * N + r), mask=lane0)
            plsc.store_scatter(svb, [splat_i(r)], splat_f(-1.0), mask=lane0)
            c1n = jnp.where(idx1 == r, -1.0, c1)
            c2n = jnp.where(idx2 == r, -1.0, c2)
            nm = jnp.maximum(jnp.max(c1n), jnp.max(c2n))
            plsc.store_scatter(gmb, [splat_i(g * 16 + L)], splat_f(nm), mask=lane0)
            colg2 = jnp.where(IOTA == g, nm, colg)
            cm = jnp.where(IOTA == L, jnp.max(colg2), cm)
            return cm

        cm = lax.fori_loop(0, K, extb_body, cm)

        pltpu.async_copy(lgt_hbm.at[gib], lrows, sem).wait()

        def pc_body(i, _):
            iv = splat_i(i)
            rvec = plsc.load_gather(rkb, [iv])
            ov = plsc.load_gather(ovb, [rvec])
            eo = jnp.exp(-ov)
            acc = splat_f(-2.0)
            for off in (0, 16, 32, 48, 64, 80):
                x = plsc.load_gather(lrows, [iv, off + IOTA])
                p = eo * (1.0 / (1.0 + jnp.exp(-x)))
                if off + 16 > NUM_VALID:
                    p = jnp.where(off + IOTA < NUM_VALID, p, -1.0)
                plsc.store_scatter(pmat, [iv, off + IOTA], p)
                acc = jnp.maximum(acc, p)
            racc = jnp.max(acc)
            plsc.store_scatter(rvb, [iv], splat_f(racc), mask=lane0)
            return 0

        lax.fori_loop(0, K, pc_body, 0)

        def vm_body(j, acc):
            return jnp.maximum(acc, plsc.load_gather(rvb, [j * 16 + IOTA]))

        vm = lax.fori_loop(0, KP // 16, vm_body, splat_f(-2.0))

        plsc.store_scatter(rloc, [96 + IOTA], splat_i(0))

        def extc_body(k, vm):
            mval = jnp.max(vm)
            L = jnp.min(jnp.where(vm == mval, IOTA, BIG))
            validc = IOTA < (KP // 16)
            colv = plsc.load_gather(rvb, [jnp.where(validc, IOTA * 16 + L, 0)])
            colv = jnp.where(validc, colv, -2.0)
            jj = jnp.min(jnp.where(colv == mval, IOTA, BIG))
            istar = jj * 16 + L
            iv = splat_i(istar)
            cls_acc = jnp.int32(BIG)
            pvs = []
            for off in (0, 16, 32, 48, 64, 80):
                p = plsc.load_gather(pmat, [iv, off + IOTA])
                cls_acc = jnp.minimum(
                    cls_acc, jnp.min(jnp.where(p == mval, off + IOTA, BIG))
                )
                pvs.append(p)
            nrm = jnp.float32(-2.0)
            for off, p in zip((0, 16, 32, 48, 64, 80), pvs):
                pn = jnp.where(off + IOTA == cls_acc, -1.0, p)
                nrm = jnp.maximum(nrm, jnp.max(pn))
            plsc.store_scatter(pmat, [iv, splat_i(cls_acc)], splat_f(-1.0),
                               mask=lane0)
            plsc.store_scatter(rvb, [iv], splat_f(nrm), mask=lane0)
            kv = splat_i(k)
            plsc.store_scatter(scob, [kv], splat_f(mval), mask=lane0)
            plsc.store_scatter(labb, [kv], splat_i(cls_acc), mask=lane0)
            rv2 = plsc.load_gather(rkb, [iv])
            plsc.store_scatter(rloc, [kv], rv2, mask=lane0)
            colv2 = jnp.where(IOTA == jj, nrm, colv)
            vm = jnp.where(IOTA == L, jnp.max(colv2), vm)
            return vm

        vm = lax.fori_loop(0, K, extc_body, vm)

        def pd_body(v, _):
            pos = v * 16 + IOTA
            obx = pos >> 2
            comp = pos & 3
            r = plsc.load_gather(rloc, [obx])
            c = plsc.load_gather(bxlb, [r * 4 + (comp & 1)])
            w = plsc.load_gather(bxlb, [r * 4 + 2 + (comp & 1)])
            sgn = jnp.where(comp < 2, jnp.float32(-0.5), jnp.float32(0.5))
            scl = plsc.load_gather(sclvb, [comp])
            plsc.store_scatter(bxob, [pos], (c + sgn * w) * scl)
            return 0

        lax.fori_loop(0, (KP * 4) // 16, pd_body, 0)

        pltpu.sync_copy(scob, outs.at[pl.ds(b * KP, KP)])
        pltpu.sync_copy(labb, outl.at[pl.ds(b * KP, KP)])
        pltpu.sync_copy(bxob, outb.at[pl.ds(b * KP * 4, KP * 4)])


def _sc_phase(m_p, ob_p, lgt, bx, scl):
    f32 = jnp.float32
    fn = pl.kernel(
        _postprocess_sc,
        out_type=[
            jax.ShapeDtypeStruct((B * KP,), f32),
            jax.ShapeDtypeStruct((B * KP,), jnp.int32),
            jax.ShapeDtypeStruct((B * KP * 4,), f32),
        ],
        mesh=plsc.VectorSubcoreMesh(core_axis_name="c", subcore_axis_name="s"),
        scratch_types=[
            pltpu.VMEM((NP,), f32),
            pltpu.VMEM((NP,), f32),
            pltpu.VMEM((NGRP * 16,), f32),
            pltpu.VMEM((KP,), jnp.int32),
            pltpu.VMEM((KP,), f32),
            pltpu.VMEM((KP,), jnp.int32),
            pltpu.VMEM((KP, 128), f32),
            pltpu.VMEM((KP, 96), f32),
            pltpu.VMEM((KP,), jnp.int32),
            pltpu.VMEM((KP,), f32),
            pltpu.VMEM((KP,), jnp.int32),
            pltpu.VMEM((KP * 4,), f32),
            pltpu.VMEM((4 * N,), f32),
            pltpu.VMEM((16,), f32),
            pltpu.SemaphoreType.DMA,
        ],
        compiler_params=pltpu.CompilerParams(needs_layout_passes=False),
    )
    return fn(m_p, ob_p, lgt, bx, scl)


def kernel(pred_logits, pred_obj, pred_boxes, target_sizes):
    m, lgp = _rowmax(pred_logits)
    m_p = jnp.pad(m.reshape(B, N), ((0, 0), (0, NP - N))).reshape(-1)
    ob_p = jnp.pad(pred_obj, ((0, 0), (0, NP - N))).reshape(-1)
    lgt = lgp.reshape(B * N, 128)
    bx = pred_boxes.reshape(-1)
    scl = jnp.tile(
        jnp.stack(
            [target_sizes[:, 1], target_sizes[:, 0],
             target_sizes[:, 1], target_sizes[:, 0]], axis=1),
        (1, 4),
    ).reshape(-1)
    outs, outl, outb = _sc_phase(m_p, ob_p, lgt, bx, scl)
    scores = outs.reshape(B, KP)[:, :K]
    labels = outl.reshape(B, KP)[:, :K]
    boxes = outb.reshape(B, KP, 4)[:, :K, :]
    return scores, labels, boxes


# speedup vs baseline: 6.6923x; 1.1121x over previous
import functools

import jax
import jax.numpy as jnp
from jax import lax
from jax.experimental import pallas as pl
from jax.experimental.pallas import tpu as pltpu
from jax.experimental.pallas import tpu_sc as plsc

B = 8
N = 5000
C = 91
NUM_VALID = 81
K = 100
RB = 5000
NP = 5120
NVR = NP // 16
NGRP = 16
GV = NVR // NGRP
KP = 112
BIG = 1 << 20


def _rowmax_body(lg_ref, m_ref, lgp_ref):
    x = lg_ref[0]
    m_ref[0, 0] = jnp.max(x[:, :NUM_VALID], axis=1)
    lgp_ref[0, :, :C] = x


def _rowmax(pred_logits):
    return pl.pallas_call(
        _rowmax_body,
        grid=(B, N // RB),
        in_specs=[pl.BlockSpec((1, RB, C), lambda b, i: (b, i, 0))],
        out_specs=[
            pl.BlockSpec((1, 1, RB), lambda b, i: (b * (N // RB) + i, 0, 0)),
            pl.BlockSpec((1, RB, 128), lambda b, i: (b * (N // RB) + i, 0, 0)),
        ],
        out_shape=[
            jax.ShapeDtypeStruct((B * N // RB, 1, RB), jnp.float32),
            jax.ShapeDtypeStruct((B * N // RB, RB, 128), jnp.float32),
        ],
    )(pred_logits)


def _postprocess_sc(m_hbm, ob_hbm, lgt_hbm, bx_hbm, scl_hbm,
                    outs, outl, outb,
                    svb, ovb, gmb, rkb, rvb, gib, lrows, pmat, rloc,
                    scob, labb, bxob, bxlb, sclvb, sem):
    wid = lax.axis_index("s") * 2 + lax.axis_index("c")

    @pl.when(wid < B)
    def _():
        b = wid
        IOTA = lax.iota(jnp.int32, 16)

        def splat_f(x):
            return jnp.full((16,), x, jnp.float32)

        def splat_i(x):
            return jnp.full((16,), x, jnp.int32)

        lane0 = IOTA == 0

        pltpu.sync_copy(m_hbm.at[pl.ds(b * NP, NP)], svb)
        pltpu.sync_copy(ob_hbm.at[pl.ds(b * NP, NP)], ovb)
        pltpu.sync_copy(bx_hbm.at[pl.ds(b * (4 * N), 4 * N)], bxlb)
        pltpu.sync_copy(scl_hbm.at[pl.ds(b * 16, 16)], sclvb)

        def init_body(i, _):
            idx = i * 16 + IOTA
            mv = plsc.load_gather(svb, [idx])
            ov = plsc.load_gather(ovb, [idx])
            val = jnp.exp(-ov) * (1.0 / (1.0 + jnp.exp(-mv)))
            val = jnp.where(idx < N, val, -1.0)
            plsc.store_scatter(svb, [idx], val)
            return 0

        lax.fori_loop(0, NVR, init_body, 0)

        def gm_body(g, _):
            def inner(j, acc):
                return jnp.maximum(
                    acc, plsc.load_gather(svb, [(g * GV + j) * 16 + IOTA])
                )

            acc = lax.fori_loop(0, GV, inner, splat_f(-2.0))
            plsc.store_scatter(gmb, [g * 16 + IOTA], acc)
            return 0

        lax.fori_loop(0, NGRP, gm_body, 0)

        def cm_body(g, acc):
            return jnp.maximum(acc, plsc.load_gather(gmb, [g * 16 + IOTA]))

        cm = lax.fori_loop(0, NGRP, cm_body, splat_f(-2.0))

        plsc.store_scatter(rvb, [96 + IOTA], splat_f(-1.0))
        plsc.store_scatter(rkb, [96 + IOTA], splat_i(0))
        plsc.store_scatter(gib, [96 + IOTA], splat_i(b * N))

        def extb_body(k, cm):
            mval = jnp.max(cm)
            L = jnp.min(jnp.where(cm == mval, IOTA, BIG))
            colg = plsc.load_gather(gmb, [IOTA * 16 + L])
            g = jnp.min(jnp.where(colg == mval, IOTA, BIG))
            base = g * GV * 16 + L
            idx1 = base + IOTA * 16
            valid2 = IOTA < (GV - 16)
            idx2 = base + (16 + IOTA) * 16
            c1 = plsc.load_gather(svb, [idx1])
            c2 = plsc.load_gather(svb, [jnp.where(valid2, idx2, 0)])
            c2 = jnp.where(valid2, c2, -2.0)
            j1 = jnp.min(jnp.where(c1 == mval, IOTA, BIG))
            j2 = jnp.min(jnp.where(c2 == mval, 16 + IOTA, BIG))
            j = jnp.minimum(j1, j2)
            r = (g * GV + j) * 16 + L
            plsc.store_scatter(rkb, [splat_i(k)], splat_i(r), mask=lane0)
            plsc.store_scatter(rvb, [splat_i(k)], splat_f(mval), mask=lane0)
            plsc.store_scatter(gib, [splat_i(k)], splat_i(b * N + r), mask=lane0)
            plsc.store_scatter(svb, [splat_i(r)], splat_f(-1.0), mask=lane0)
            c1n = jnp.where(idx1 == r, -1.0, c1)
            c2n = jnp.where(idx2 == r, -1.0, c2)
            nm = jnp.maximum(jnp.max(c1n), jnp.max(c2n))
            plsc.store_scatter(gmb, [splat_i(g * 16 + L)], splat_f(nm), mask=lane0)
            colg2 = jnp.where(IOTA == g, nm, colg)
            cm = jnp.where(IOTA == L, jnp.max(colg2), cm)
            return cm

        cm = lax.fori_loop(0, K, extb_body, cm)

        pltpu.async_copy(lgt_hbm.at[gib], lrows, sem).wait()

        def pc_body(i, _):
            iv = splat_i(i)
            rvec = plsc.load_gather(rkb, [iv])
            ov = plsc.load_gather(ovb, [rvec])
            eo = jnp.exp(-ov)
            acc = splat_f(-2.0)
            for off in (0, 16, 32, 48, 64, 80):
                x = plsc.load_gather(lrows, [iv, off + IOTA])
                p = eo * (1.0 / (1.0 + jnp.exp(-x)))
                if off + 16 > NUM_VALID:
                    p = jnp.where(off + IOTA < NUM_VALID, p, -1.0)
                plsc.store_scatter(pmat, [iv, off + IOTA], p)
                acc = jnp.maximum(acc, p)
            racc = jnp.max(acc)
            plsc.store_scatter(rvb, [iv], splat_f(racc), mask=lane0)
            return 0

        lax.fori_loop(0, K, pc_body, 0)

        def vm_body(j, acc):
            return jnp.maximum(acc, plsc.load_gather(rvb, [j * 16 + IOTA]))

        vm = lax.fori_loop(0, KP // 16, vm_body, splat_f(-2.0))

        plsc.store_scatter(rloc, [96 + IOTA], splat_i(0))

        def extc_body(k, vm):
            mval = jnp.max(vm)
            L = jnp.min(jnp.where(vm == mval, IOTA, BIG))
            validc = IOTA < (KP // 16)
            colv = plsc.load_gather(rvb, [jnp.where(validc, IOTA * 16 + L, 0)])
            colv = jnp.where(validc, colv, -2.0)
            jj = jnp.min(jnp.where(colv == mval, IOTA, BIG))
            istar = jj * 16 + L
            iv = splat_i(istar)
            cls_acc = jnp.int32(BIG)
            pvs = []
            for off in (0, 16, 32, 48, 64, 80):
                p = plsc.load_gather(pmat, [iv, off + IOTA])
                cls_acc = jnp.minimum(
                    cls_acc, jnp.min(jnp.where(p == mval, off + IOTA, BIG))
                )
                pvs.append(p)
            nrm = jnp.float32(-2.0)
            for off, p in zip((0, 16, 32, 48, 64, 80), pvs):
                pn = jnp.where(off + IOTA == cls_acc, -1.0, p)
                nrm = jnp.maximum(nrm, jnp.max(pn))
            plsc.store_scatter(pmat, [iv, splat_i(cls_acc)], splat_f(-1.0),
                               mask=lane0)
            plsc.store_scatter(rvb, [iv], splat_f(nrm), mask=lane0)
            kv = splat_i(k)
            plsc.store_scatter(scob, [kv], splat_f(mval), mask=lane0)
            plsc.store_scatter(labb, [kv], splat_i(cls_acc), mask=lane0)
            rv2 = plsc.load_gather(rkb, [iv])
            plsc.store_scatter(rloc, [kv], rv2, mask=lane0)
            colv2 = jnp.where(IOTA == jj, nrm, colv)
            vm = jnp.where(IOTA == L, jnp.max(colv2), vm)
            return vm

        vm = lax.fori_loop(0, K, extc_body, vm)

        def pd_body(v, _):
            pos = v * 16 + IOTA
            obx = pos >> 2
            comp = pos & 3
            r = plsc.load_gather(rloc, [obx])
            c = plsc.load_gather(bxlb, [r * 4 + (comp & 1)])
            w = plsc.load_gather(bxlb, [r * 4 + 2 + (comp & 1)])
            sgn = jnp.where(comp < 2, jnp.float32(-0.5), jnp.float32(0.5))
            scl = plsc.load_gather(sclvb, [comp])
            plsc.store_scatter(bxob, [pos], (c + sgn * w) * scl)
            return 0

        lax.fori_loop(0, (KP * 4) // 16, pd_body, 0)

        pltpu.sync_copy(scob, outs.at[pl.ds(b * KP, KP)])
        pltpu.sync_copy(labb, outl.at[pl.ds(b * KP, KP)])
        pltpu.sync_copy(bxob, outb.at[pl.ds(b * KP * 4, KP * 4)])


def _sc_phase(m_p, ob_p, lgt, bx, scl):
    f32 = jnp.float32
    fn = pl.kernel(
        _postprocess_sc,
        out_type=[
            jax.ShapeDtypeStruct((B * KP,), f32),
            jax.ShapeDtypeStruct((B * KP,), jnp.int32),
            jax.ShapeDtypeStruct((B * KP * 4,), f32),
        ],
        mesh=plsc.VectorSubcoreMesh(core_axis_name="c", subcore_axis_name="s"),
        scratch_types=[
            pltpu.VMEM((NP,), f32),
            pltpu.VMEM((NP,), f32),
            pltpu.VMEM((NGRP * 16,), f32),
            pltpu.VMEM((KP,), jnp.int32),
            pltpu.VMEM((KP,), f32),
            pltpu.VMEM((KP,), jnp.int32),
            pltpu.VMEM((KP, 128), f32),
            pltpu.VMEM((KP, 96), f32),
            pltpu.VMEM((KP,), jnp.int32),
            pltpu.VMEM((KP,), f32),
            pltpu.VMEM((KP,), jnp.int32),
            pltpu.VMEM((KP * 4,), f32),
            pltpu.VMEM((4 * N,), f32),
            pltpu.VMEM((16,), f32),
            pltpu.SemaphoreType.DMA,
        ],
        compiler_params=pltpu.CompilerParams(needs_layout_passes=False),
    )
    return fn(m_p, ob_p, lgt, bx, scl)


def kernel(pred_logits, pred_obj, pred_boxes, target_sizes):
    m, lgp = _rowmax(pred_logits)
    m_p = jnp.pad(m.reshape(B, N), ((0, 0), (0, NP - N))).reshape(-1)
    ob_p = jnp.pad(pred_obj, ((0, 0), (0, NP - N))).reshape(-1)
    lgt = lgp.reshape(B * N, 128)
    bx = pred_boxes.reshape(-1)
    scl = jnp.tile(
        jnp.stack(
            [target_sizes[:, 1], target_sizes[:, 0],
             target_sizes[:, 1], target_sizes[:, 0]], axis=1),
        (1, 4),
    ).reshape(-1)
    outs, outl, outb = _sc_phase(m_p, ob_p, lgt, bx, scl)
    scores = outs.reshape(B, KP)[:, :K]
    labels = outl.reshape(B, KP)[:, :K]
    boxes = outb.reshape(B, KP, 4)[:, :K, :]
    return scores, labels, boxes
